# baseline (speedup 1.0000x reference)
import functools

import jax
import jax.numpy as jnp
from jax import lax
from jax.experimental import pallas as pl
from jax.experimental.pallas import tpu as pltpu
from jax.experimental.pallas import tpu_sc as plsc

NUM_HEADS = 8
HEAD_DIM = 16
DIM = NUM_HEADS * HEAD_DIM

NC = 2
NS = 16
NW = NC * NS

CHUNK = 80


def _mm_body(x_ref, w_ref, o_ref):
    m = lax.dot_general(x_ref[...], w_ref[...], (((2,), (0,)), ((), ())),
                        preferred_element_type=jnp.float32)
    o_ref[...] = m.reshape(o_ref.shape)


def _score_body(msg_ref, w_ref, i_ref, g_ref, v_ref, mt_ref):
    msg = msg_ref[...]
    m = lax.dot_general(msg, w_ref[...], (((2,), (0,)), ((), ())),
                        preferred_element_type=jnp.float32)
    x = m.reshape(g_ref.shape) + g_ref[...]
    x = jnp.maximum(x, 0.01 * x)
    v_ref[...] = jnp.exp(x)
    msg2 = msg.reshape(msg.shape[0] * HEAD_DIM, DIM)
    nrow = msg2.shape[0]
    for t in range(nrow // DIM):
        blk = lax.slice(msg2, (t * DIM, 0), ((t + 1) * DIM, DIM))
        mt_ref[:, pl.ds(t * DIM, DIM)] = lax.dot_general(
            blk, i_ref[...], (((0,), (0,)), ((), ())),
            precision=lax.Precision.HIGHEST,
            preferred_element_type=jnp.float32)


def _add_body(p_ref, o_ref):
    o_ref[...] = p_ref[0] + p_ref[1]


def _div_body(v_ref, d_ref, o_ref):
    a = v_ref[...] / (d_ref[...] + 1e-16)
    rb = a.shape[0]
    a4 = a.reshape(rb // 8, 8, HEAD_DIM, NUM_HEADS)
    o_ref[...] = a4.transpose(3, 0, 1, 2).reshape(NUM_HEADS, rb * HEAD_DIM)


def kernel(edge_index, message, x_e, weight):
    num_edges = message.shape[0]
    num_nodes = x_e.shape[0]
    target = edge_index[1]

    w_m = weight[:, :HEAD_DIM]
    w_x = weight[:, HEAD_DIM:]
    eye = jnp.eye(NUM_HEADS, dtype=jnp.float32)
    W1 = (w_m[:, :, None] * eye[:, None, :]).reshape(DIM, NUM_HEADS)
    W2 = (w_x[:, :, None] * eye[:, None, :]).reshape(DIM, NUM_HEADS)

    n_chunks = num_edges // CHUNK
    rpw = n_chunks // NW
    nodes_pad = ((num_nodes + 16 * NS - 1) // (16 * NS)) * (16 * NS)
    rows_per_tile = nodes_pad // NS
    lanes = 128
    eflat_rows = num_edges * NUM_HEADS // lanes
    nflat_rows = num_nodes * NUM_HEADS // lanes

    t3d = target.reshape(NW, rpw, CHUNK)

    s_flat = pl.pallas_call(
        _mm_body,
        in_specs=[pl.BlockSpec((nflat_rows, HEAD_DIM, DIM), lambda: (0, 0, 0)),
                  pl.BlockSpec((DIM, NUM_HEADS), lambda: (0, 0))],
        out_specs=pl.BlockSpec((nflat_rows, lanes), lambda: (0, 0)),
        out_shape=jax.ShapeDtypeStruct((nflat_rows, lanes), jnp.float32),
    )(x_e.reshape(nflat_rows, HEAD_DIM, DIM), W2)
    s = s_flat.reshape(num_nodes, NUM_HEADS)

    mesh = plsc.VectorSubcoreMesh(core_axis_name="c", subcore_axis_name="s")

    @functools.partial(
        pl.kernel, mesh=mesh,
        compiler_params=pltpu.CompilerParams(use_tc_tiling_on_sc=False),
        out_type=jax.ShapeDtypeStruct((n_chunks, CHUNK, NUM_HEADS),
                                      jnp.float32),
        scratch_types=[
            pltpu.VMEM((rpw, CHUNK), jnp.int32),
            pltpu.VMEM((rpw, CHUNK, NUM_HEADS), jnp.float32),
            pltpu.SemaphoreType.DMA,
        ],
    )
    def gather_k(s_hbm, t_hbm, g_hbm, idx_v, big_v, sem):
        wid = lax.axis_index("c") * NS + lax.axis_index("s")
        row0 = wid * rpw
        pltpu.sync_copy(t_hbm.at[wid], idx_v)

        def body(j, carry):
            pltpu.async_copy(s_hbm.at[idx_v.at[j]], big_v.at[j], sem)
            return carry

        lax.fori_loop(0, rpw, body, 0)
        pltpu.make_async_copy(g_hbm.at[pl.ds(row0, rpw)], big_v, sem).wait()
        pltpu.sync_copy(big_v, g_hbm.at[pl.ds(row0, rpw)])

    g_flat = gather_k(s, t3d).reshape(eflat_rows, lanes)

    eb = 2560
    fb = eb * NUM_HEADS // lanes
    ident = jnp.eye(DIM, dtype=jnp.float32)
    v_flat, msg_t = pl.pallas_call(
        _score_body,
        grid=(num_edges // eb,),
        in_specs=[pl.BlockSpec((fb, HEAD_DIM, DIM), lambda i: (i, 0, 0)),
                  pl.BlockSpec((DIM, NUM_HEADS), lambda i: (0, 0)),
                  pl.BlockSpec((DIM, DIM), lambda i: (0, 0)),
                  pl.BlockSpec((fb, lanes), lambda i: (i, 0))],
        out_specs=[pl.BlockSpec((fb, lanes), lambda i: (i, 0)),
                   pl.BlockSpec((DIM, eb), lambda i: (0, i))],
        out_shape=[jax.ShapeDtypeStruct((eflat_rows, lanes), jnp.float32),
                   jax.ShapeDtypeStruct((DIM, num_edges), jnp.float32)],
    )(message.reshape(eflat_rows, HEAD_DIM, DIM), W1, ident, g_flat)

    v3d = v_flat.reshape(n_chunks, CHUNK, NUM_HEADS)

    zeros_tab = jnp.zeros((nodes_pad, NUM_HEADS), jnp.float32)

    @functools.partial(
        pl.kernel, mesh=mesh,
        compiler_params=pltpu.CompilerParams(use_tc_tiling_on_sc=False),
        out_type=jax.ShapeDtypeStruct((NC, nodes_pad, NUM_HEADS), jnp.float32),
        scratch_types=[
            pltpu.VMEM((rpw, CHUNK), jnp.int32),
            pltpu.VMEM((rpw, CHUNK, NUM_HEADS), jnp.float32),
            pltpu.VMEM_SHARED((nodes_pad, NUM_HEADS), jnp.float32),
            pltpu.SemaphoreType.DMA,
            pltpu.SemaphoreType.DMA,
        ],
    )
    def segsum_k(v_hbm, t_hbm, z_hbm, part_hbm, idx_v, big_v, seg_sh,
                 sem, sem2):
        c = lax.axis_index("c")
        sid = lax.axis_index("s")
        wid = c * NS + sid
        row0 = wid * rpw

        pltpu.sync_copy(z_hbm.at[pl.ds(sid * rows_per_tile, rows_per_tile)],
                        seg_sh.at[pl.ds(sid * rows_per_tile, rows_per_tile)])
        pltpu.sync_copy(t_hbm.at[wid], idx_v)
        pltpu.sync_copy(v_hbm.at[pl.ds(row0, rpw)], big_v)
        plsc.subcore_barrier()

        def body(j, carry):
            pltpu.async_copy(big_v.at[j], seg_sh.at[idx_v.at[j]], sem2,
                             add=True)
            return carry

        lax.fori_loop(0, rpw, body, 0)
        pltpu.make_async_copy(v_hbm.at[pl.ds(row0, rpw)], big_v, sem2).wait()
        plsc.subcore_barrier()

        pltpu.sync_copy(seg_sh.at[pl.ds(sid * rows_per_tile, rows_per_tile)],
                        part_hbm.at[c, pl.ds(sid * rows_per_tile,
                                             rows_per_tile)])

    partial_tabs = segsum_k(v3d, t3d, zeros_tab)

    tab_rows = nodes_pad * NUM_HEADS // lanes
    seg_flat = pl.pallas_call(
        _add_body,
        in_specs=[pl.BlockSpec((NC, tab_rows, lanes), lambda: (0, 0, 0))],
        out_specs=pl.BlockSpec((tab_rows, lanes), lambda: (0, 0)),
        out_shape=jax.ShapeDtypeStruct((tab_rows, lanes), jnp.float32),
    )(partial_tabs.reshape(NC, tab_rows, lanes))
    seg_tab = seg_flat.reshape(nodes_pad, NUM_HEADS)

    @functools.partial(
        pl.kernel, mesh=mesh,
        compiler_params=pltpu.CompilerParams(use_tc_tiling_on_sc=False),
        out_type=jax.ShapeDtypeStruct((n_chunks, CHUNK, NUM_HEADS),
                                      jnp.float32),
        scratch_types=[
            pltpu.VMEM((rpw, CHUNK), jnp.int32),
            pltpu.VMEM((rpw, CHUNK, NUM_HEADS), jnp.float32),
            pltpu.SemaphoreType.DMA,
        ],
    )
    def denom_k(p_hbm, t_hbm, d_hbm, idx_v, big_v, sem):
        wid = lax.axis_index("c") * NS + lax.axis_index("s")
        row0 = wid * rpw
        pltpu.sync_copy(t_hbm.at[wid], idx_v)

        def body(j, carry):
            pltpu.async_copy(p_hbm.at[idx_v.at[j]], big_v.at[j], sem)
            return carry

        lax.fori_loop(0, rpw, body, 0)
        pltpu.make_async_copy(d_hbm.at[pl.ds(row0, rpw)], big_v, sem).wait()
        pltpu.sync_copy(big_v, d_hbm.at[pl.ds(row0, rpw)])

    d_flat = denom_k(seg_tab, t3d).reshape(eflat_rows, lanes)

    db = 2000
    alpha_t = pl.pallas_call(
        _div_body,
        grid=(eflat_rows // db,),
        in_specs=[pl.BlockSpec((db, lanes), lambda i: (i, 0))] * 2,
        out_specs=pl.BlockSpec((NUM_HEADS, db * lanes // NUM_HEADS),
                               lambda i: (0, i)),
        out_shape=jax.ShapeDtypeStruct((NUM_HEADS, num_edges), jnp.float32),
    )(v_flat, d_flat)

    alpha = alpha_t.T
    message_ = msg_t.reshape(NUM_HEADS, HEAD_DIM, num_edges).transpose(2, 0, 1)
    return message_, alpha


# speedup vs baseline: 1.1084x; 1.1084x over previous
import functools

import jax
import jax.numpy as jnp
from jax import lax
from jax.experimental import pallas as pl
from jax.experimental.pallas import tpu as pltpu
from jax.experimental.pallas import tpu_sc as plsc

NUM_HEADS = 8
HEAD_DIM = 16
DIM = NUM_HEADS * HEAD_DIM

NC = 2
NS = 16
NW = NC * NS

CHUNK = 80


def _mm_body(x_ref, w_ref, o_ref):
    m = lax.dot_general(x_ref[...], w_ref[...], (((2,), (0,)), ((), ())),
                        preferred_element_type=jnp.float32)
    o_ref[...] = m.reshape(o_ref.shape)


def _score_body(msg_ref, w_ref, g_ref, v_ref, mt_ref):
    msg = msg_ref[...]
    m = lax.dot_general(msg, w_ref[...], (((2,), (0,)), ((), ())),
                        preferred_element_type=jnp.float32)
    x = m.reshape(g_ref.shape) + g_ref[...]
    x = jnp.maximum(x, 0.01 * x)
    v_ref[...] = jnp.exp(x)
    msg2 = msg.reshape(msg.shape[0] * HEAD_DIM, DIM)
    mt_ref[...] = msg2.T


def _add_body(p_ref, o_ref):
    o_ref[...] = p_ref[0] + p_ref[1]


def _div_body(v_ref, d_ref, o_ref):
    a = v_ref[...] / (d_ref[...] + 1e-16)
    rb = a.shape[0]
    a4 = a.reshape(rb // 8, 8, HEAD_DIM, NUM_HEADS)
    o_ref[...] = a4.transpose(3, 0, 1, 2).reshape(NUM_HEADS, rb * HEAD_DIM)


def kernel(edge_index, message, x_e, weight):
    num_edges = message.shape[0]
    num_nodes = x_e.shape[0]
    target = edge_index[1]

    w_m = weight[:, :HEAD_DIM]
    w_x = weight[:, HEAD_DIM:]
    eye = jnp.eye(NUM_HEADS, dtype=jnp.float32)
    W1 = (w_m[:, :, None] * eye[:, None, :]).reshape(DIM, NUM_HEADS)
    W2 = (w_x[:, :, None] * eye[:, None, :]).reshape(DIM, NUM_HEADS)

    n_chunks = num_edges // CHUNK
    rpw = n_chunks // NW
    nodes_pad = ((num_nodes + 16 * NS - 1) // (16 * NS)) * (16 * NS)
    rows_per_tile = nodes_pad // NS
    lanes = 128
    eflat_rows = num_edges * NUM_HEADS // lanes
    nflat_rows = num_nodes * NUM_HEADS // lanes

    t3d = target.reshape(NW, rpw, CHUNK)

    s_flat = pl.pallas_call(
        _mm_body,
        in_specs=[pl.BlockSpec((nflat_rows, HEAD_DIM, DIM), lambda: (0, 0, 0)),
                  pl.BlockSpec((DIM, NUM_HEADS), lambda: (0, 0))],
        out_specs=pl.BlockSpec((nflat_rows, lanes), lambda: (0, 0)),
        out_shape=jax.ShapeDtypeStruct((nflat_rows, lanes), jnp.float32),
    )(x_e.reshape(nflat_rows, HEAD_DIM, DIM), W2)
    s = s_flat.reshape(num_nodes, NUM_HEADS)

    mesh = plsc.VectorSubcoreMesh(core_axis_name="c", subcore_axis_name="s")

    @functools.partial(
        pl.kernel, mesh=mesh,
        compiler_params=pltpu.CompilerParams(use_tc_tiling_on_sc=False),
        out_type=jax.ShapeDtypeStruct((n_chunks, CHUNK, NUM_HEADS),
                                      jnp.float32),
        scratch_types=[
            pltpu.VMEM((rpw, CHUNK), jnp.int32),
            pltpu.VMEM((rpw, CHUNK, NUM_HEADS), jnp.float32),
            pltpu.SemaphoreType.DMA,
        ],
    )
    def gather_k(s_hbm, t_hbm, g_hbm, idx_v, big_v, sem):
        wid = lax.axis_index("c") * NS + lax.axis_index("s")
        row0 = wid * rpw
        pltpu.sync_copy(t_hbm.at[wid], idx_v)

        def body(j, carry):
            pltpu.async_copy(s_hbm.at[idx_v.at[j]], big_v.at[j], sem)
            return carry

        lax.fori_loop(0, rpw, body, 0)
        pltpu.make_async_copy(g_hbm.at[pl.ds(row0, rpw)], big_v, sem).wait()
        pltpu.sync_copy(big_v, g_hbm.at[pl.ds(row0, rpw)])

    g_flat = gather_k(s, t3d).reshape(eflat_rows, lanes)

    eb = 2560
    fb = eb * NUM_HEADS // lanes
    v_flat, msg_t = pl.pallas_call(
        _score_body,
        grid=(num_edges // eb,),
        in_specs=[pl.BlockSpec((fb, HEAD_DIM, DIM), lambda i: (i, 0, 0)),
                  pl.BlockSpec((DIM, NUM_HEADS), lambda i: (0, 0)),
                  pl.BlockSpec((fb, lanes), lambda i: (i, 0))],
        out_specs=[pl.BlockSpec((fb, lanes), lambda i: (i, 0)),
                   pl.BlockSpec((DIM, eb), lambda i: (0, i))],
        out_shape=[jax.ShapeDtypeStruct((eflat_rows, lanes), jnp.float32),
                   jax.ShapeDtypeStruct((DIM, num_edges), jnp.float32)],
    )(message.reshape(eflat_rows, HEAD_DIM, DIM), W1, g_flat)

    v3d = v_flat.reshape(n_chunks, CHUNK, NUM_HEADS)

    zeros_tab = jnp.zeros((nodes_pad, NUM_HEADS), jnp.float32)

    @functools.partial(
        pl.kernel, mesh=mesh,
        compiler_params=pltpu.CompilerParams(use_tc_tiling_on_sc=False),
        out_type=jax.ShapeDtypeStruct((NC, nodes_pad, NUM_HEADS), jnp.float32),
        scratch_types=[
            pltpu.VMEM((rpw, CHUNK), jnp.int32),
            pltpu.VMEM((rpw, CHUNK, NUM_HEADS), jnp.float32),
            pltpu.VMEM_SHARED((nodes_pad, NUM_HEADS), jnp.float32),
            pltpu.SemaphoreType.DMA,
            pltpu.SemaphoreType.DMA,
        ],
    )
    def segsum_k(v_hbm, t_hbm, z_hbm, part_hbm, idx_v, big_v, seg_sh,
                 sem, sem2):
        c = lax.axis_index("c")
        sid = lax.axis_index("s")
        wid = c * NS + sid
        row0 = wid * rpw

        pltpu.sync_copy(z_hbm.at[pl.ds(sid * rows_per_tile, rows_per_tile)],
                        seg_sh.at[pl.ds(sid * rows_per_tile, rows_per_tile)])
        pltpu.sync_copy(t_hbm.at[wid], idx_v)
        pltpu.sync_copy(v_hbm.at[pl.ds(row0, rpw)], big_v)
        plsc.subcore_barrier()

        def body(j, carry):
            pltpu.async_copy(big_v.at[j], seg_sh.at[idx_v.at[j]], sem2,
                             add=True)
            return carry

        lax.fori_loop(0, rpw, body, 0)
        pltpu.make_async_copy(v_hbm.at[pl.ds(row0, rpw)], big_v, sem2).wait()
        plsc.subcore_barrier()

        pltpu.sync_copy(seg_sh.at[pl.ds(sid * rows_per_tile, rows_per_tile)],
                        part_hbm.at[c, pl.ds(sid * rows_per_tile,
                                             rows_per_tile)])

    partial_tabs = segsum_k(v3d, t3d, zeros_tab)

    tab_rows = nodes_pad * NUM_HEADS // lanes
    seg_flat = pl.pallas_call(
        _add_body,
        in_specs=[pl.BlockSpec((NC, tab_rows, lanes), lambda: (0, 0, 0))],
        out_specs=pl.BlockSpec((tab_rows, lanes), lambda: (0, 0)),
        out_shape=jax.ShapeDtypeStruct((tab_rows, lanes), jnp.float32),
    )(partial_tabs.reshape(NC, tab_rows, lanes))
    seg_tab = seg_flat.reshape(nodes_pad, NUM_HEADS)

    @functools.partial(
        pl.kernel, mesh=mesh,
        compiler_params=pltpu.CompilerParams(use_tc_tiling_on_sc=False),
        out_type=jax.ShapeDtypeStruct((n_chunks, CHUNK, NUM_HEADS),
                                      jnp.float32),
        scratch_types=[
            pltpu.VMEM((rpw, CHUNK), jnp.int32),
            pltpu.VMEM((rpw, CHUNK, NUM_HEADS), jnp.float32),
            pltpu.SemaphoreType.DMA,
        ],
    )
    def denom_k(p_hbm, t_hbm, d_hbm, idx_v, big_v, sem):
        wid = lax.axis_index("c") * NS + lax.axis_index("s")
        row0 = wid * rpw
        pltpu.sync_copy(t_hbm.at[wid], idx_v)

        def body(j, carry):
            pltpu.async_copy(p_hbm.at[idx_v.at[j]], big_v.at[j], sem)
            return carry

        lax.fori_loop(0, rpw, body, 0)
        pltpu.make_async_copy(d_hbm.at[pl.ds(row0, rpw)], big_v, sem).wait()
        pltpu.sync_copy(big_v, d_hbm.at[pl.ds(row0, rpw)])

    d_flat = denom_k(seg_tab, t3d).reshape(eflat_rows, lanes)

    db = 2000
    alpha_t = pl.pallas_call(
        _div_body,
        grid=(eflat_rows // db,),
        in_specs=[pl.BlockSpec((db, lanes), lambda i: (i, 0))] * 2,
        out_specs=pl.BlockSpec((NUM_HEADS, db * lanes // NUM_HEADS),
                               lambda i: (0, i)),
        out_shape=jax.ShapeDtypeStruct((NUM_HEADS, num_edges), jnp.float32),
    )(v_flat, d_flat)

    alpha = alpha_t.T
    message_ = msg_t.reshape(NUM_HEADS, HEAD_DIM, num_edges).transpose(2, 0, 1)
    return message_, alpha


# speedup vs baseline: 1.2752x; 1.1505x over previous
import functools

import jax
import jax.numpy as jnp
from jax import lax
from jax.experimental import pallas as pl
from jax.experimental.pallas import tpu as pltpu
from jax.experimental.pallas import tpu_sc as plsc

NUM_HEADS = 8
HEAD_DIM = 16
DIM = NUM_HEADS * HEAD_DIM

NC = 2
NS = 16
NW = NC * NS

CHUNK = 80


def _mm_body(x_ref, w_ref, o_ref):
    m = lax.dot_general(x_ref[...], w_ref[...], (((2,), (0,)), ((), ())),
                        preferred_element_type=jnp.float32)
    o_ref[...] = m.reshape(o_ref.shape)


def _score_body(msg_ref, w_ref, g_ref, v_ref, mt_ref):
    msg = msg_ref[...]
    m = lax.dot_general(msg, w_ref[...], (((2,), (0,)), ((), ())),
                        preferred_element_type=jnp.float32)
    x = m.reshape(g_ref.shape) + g_ref[...]
    x = jnp.maximum(x, 0.01 * x)
    v_ref[...] = jnp.exp(x)
    msg2 = msg.reshape(msg.shape[0] * HEAD_DIM, DIM)
    mt_ref[...] = msg2.T


def _add_body(p_ref, o_ref):
    o_ref[...] = p_ref[0] + p_ref[1]


def _div_body(v_ref, d_ref, o_ref):
    a = v_ref[...] / (d_ref[...] + 1e-16)
    rb = a.shape[0]
    a4 = a.reshape(rb // 8, 8, HEAD_DIM, NUM_HEADS)
    o_ref[...] = a4.transpose(3, 0, 1, 2).reshape(NUM_HEADS, rb * HEAD_DIM)


def kernel(edge_index, message, x_e, weight):
    num_edges = message.shape[0]
    num_nodes = x_e.shape[0]
    target = edge_index[1]

    w_m = weight[:, :HEAD_DIM]
    w_x = weight[:, HEAD_DIM:]
    eye = jnp.eye(NUM_HEADS, dtype=jnp.float32)
    W1 = (w_m[:, :, None] * eye[:, None, :]).reshape(DIM, NUM_HEADS)
    W2 = (w_x[:, :, None] * eye[:, None, :]).reshape(DIM, NUM_HEADS)

    n_chunks = num_edges // CHUNK
    rpw = n_chunks // NW
    nodes_pad = ((num_nodes + 16 * NS - 1) // (16 * NS)) * (16 * NS)
    rows_per_tile = nodes_pad // NS
    lanes = 128
    eflat_rows = num_edges * NUM_HEADS // lanes
    nflat_rows = num_nodes * NUM_HEADS // lanes

    t3d = target.reshape(NW, rpw, CHUNK)

    s_flat = pl.pallas_call(
        _mm_body,
        in_specs=[pl.BlockSpec((nflat_rows, HEAD_DIM, DIM), lambda: (0, 0, 0)),
                  pl.BlockSpec((DIM, NUM_HEADS), lambda: (0, 0))],
        out_specs=pl.BlockSpec((nflat_rows, lanes), lambda: (0, 0)),
        out_shape=jax.ShapeDtypeStruct((nflat_rows, lanes), jnp.float32),
    )(x_e.reshape(nflat_rows, HEAD_DIM, DIM), W2)
    s = s_flat.reshape(num_nodes, NUM_HEADS)

    mesh = plsc.VectorSubcoreMesh(core_axis_name="c", subcore_axis_name="s")

    @functools.partial(
        pl.kernel, mesh=mesh,
        compiler_params=pltpu.CompilerParams(use_tc_tiling_on_sc=False),
        out_type=jax.ShapeDtypeStruct((n_chunks, CHUNK, NUM_HEADS),
                                      jnp.float32),
        scratch_types=[
            pltpu.VMEM((rpw, CHUNK), jnp.int32),
            pltpu.VMEM((rpw, CHUNK, NUM_HEADS), jnp.float32),
            pltpu.SemaphoreType.DMA,
        ],
    )
    def gather_k(s_hbm, t_hbm, g_hbm, idx_v, big_v, sem):
        wid = lax.axis_index("c") * NS + lax.axis_index("s")
        row0 = wid * rpw
        pltpu.sync_copy(t_hbm.at[wid], idx_v)

        def body(j, carry):
            pltpu.async_copy(s_hbm.at[idx_v.at[j]], big_v.at[j], sem)
            return carry

        lax.fori_loop(0, rpw, body, 0)
        pltpu.make_async_copy(g_hbm.at[pl.ds(row0, rpw)], big_v, sem).wait()
        pltpu.sync_copy(big_v, g_hbm.at[pl.ds(row0, rpw)])

    g_flat = gather_k(s, t3d).reshape(eflat_rows, lanes)

    eb = 6400
    fb = eb * NUM_HEADS // lanes
    v_flat, msg_t = pl.pallas_call(
        _score_body,
        grid=(num_edges // eb,),
        in_specs=[pl.BlockSpec((fb, HEAD_DIM, DIM), lambda i: (i, 0, 0)),
                  pl.BlockSpec((DIM, NUM_HEADS), lambda i: (0, 0)),
                  pl.BlockSpec((fb, lanes), lambda i: (i, 0))],
        out_specs=[pl.BlockSpec((fb, lanes), lambda i: (i, 0)),
                   pl.BlockSpec((DIM, eb), lambda i: (0, i))],
        out_shape=[jax.ShapeDtypeStruct((eflat_rows, lanes), jnp.float32),
                   jax.ShapeDtypeStruct((DIM, num_edges), jnp.float32)],
    )(message.reshape(eflat_rows, HEAD_DIM, DIM), W1, g_flat)

    v3d = v_flat.reshape(n_chunks, CHUNK, NUM_HEADS)

    zeros_tab = jnp.zeros((nodes_pad, NUM_HEADS), jnp.float32)

    @functools.partial(
        pl.kernel, mesh=mesh,
        compiler_params=pltpu.CompilerParams(use_tc_tiling_on_sc=False),
        out_type=jax.ShapeDtypeStruct((NC, nodes_pad, NUM_HEADS), jnp.float32),
        scratch_types=[
            pltpu.VMEM((rpw, CHUNK), jnp.int32),
            pltpu.VMEM((rpw, CHUNK, NUM_HEADS), jnp.float32),
            pltpu.VMEM_SHARED((nodes_pad, NUM_HEADS), jnp.float32),
            pltpu.SemaphoreType.DMA,
            pltpu.SemaphoreType.DMA,
        ],
    )
    def segsum_k(v_hbm, t_hbm, z_hbm, part_hbm, idx_v, big_v, seg_sh,
                 sem, sem2):
        c = lax.axis_index("c")
        sid = lax.axis_index("s")
        wid = c * NS + sid
        row0 = wid * rpw

        pltpu.sync_copy(z_hbm.at[pl.ds(sid * rows_per_tile, rows_per_tile)],
                        seg_sh.at[pl.ds(sid * rows_per_tile, rows_per_tile)])
        pltpu.sync_copy(t_hbm.at[wid], idx_v)
        pltpu.sync_copy(v_hbm.at[pl.ds(row0, rpw)], big_v)
        plsc.subcore_barrier()

        def body(j, carry):
            pltpu.async_copy(big_v.at[j], seg_sh.at[idx_v.at[j]], sem2,
                             add=True)
            return carry

        lax.fori_loop(0, rpw, body, 0)
        pltpu.make_async_copy(v_hbm.at[pl.ds(row0, rpw)], big_v, sem2).wait()
        plsc.subcore_barrier()

        pltpu.sync_copy(seg_sh.at[pl.ds(sid * rows_per_tile, rows_per_tile)],
                        part_hbm.at[c, pl.ds(sid * rows_per_tile,
                                             rows_per_tile)])

    partial_tabs = segsum_k(v3d, t3d, zeros_tab)

    tab_rows = nodes_pad * NUM_HEADS // lanes
    seg_flat = pl.pallas_call(
        _add_body,
        in_specs=[pl.BlockSpec((NC, tab_rows, lanes), lambda: (0, 0, 0))],
        out_specs=pl.BlockSpec((tab_rows, lanes), lambda: (0, 0)),
        out_shape=jax.ShapeDtypeStruct((tab_rows, lanes), jnp.float32),
    )(partial_tabs.reshape(NC, tab_rows, lanes))
    seg_tab = seg_flat.reshape(nodes_pad, NUM_HEADS)

    @functools.partial(
        pl.kernel, mesh=mesh,
        compiler_params=pltpu.CompilerParams(use_tc_tiling_on_sc=False),
        out_type=jax.ShapeDtypeStruct((n_chunks, CHUNK, NUM_HEADS),
                                      jnp.float32),
        scratch_types=[
            pltpu.VMEM((rpw, CHUNK), jnp.int32),
            pltpu.VMEM((rpw, CHUNK, NUM_HEADS), jnp.float32),
            pltpu.SemaphoreType.DMA,
        ],
    )
    def denom_k(p_hbm, t_hbm, d_hbm, idx_v, big_v, sem):
        wid = lax.axis_index("c") * NS + lax.axis_index("s")
        row0 = wid * rpw
        pltpu.sync_copy(t_hbm.at[wid], idx_v)

        def body(j, carry):
            pltpu.async_copy(p_hbm.at[idx_v.at[j]], big_v.at[j], sem)
            return carry

        lax.fori_loop(0, rpw, body, 0)
        pltpu.make_async_copy(d_hbm.at[pl.ds(row0, rpw)], big_v, sem).wait()
        pltpu.sync_copy(big_v, d_hbm.at[pl.ds(row0, rpw)])

    d_flat = denom_k(seg_tab, t3d).reshape(eflat_rows, lanes)

    db = 2000
    alpha_t = pl.pallas_call(
        _div_body,
        grid=(eflat_rows // db,),
        in_specs=[pl.BlockSpec((db, lanes), lambda i: (i, 0))] * 2,
        out_specs=pl.BlockSpec((NUM_HEADS, db * lanes // NUM_HEADS),
                               lambda i: (0, i)),
        out_shape=jax.ShapeDtypeStruct((NUM_HEADS, num_edges), jnp.float32),
    )(v_flat, d_flat)

    alpha = alpha_t.T
    message_ = msg_t.reshape(NUM_HEADS, HEAD_DIM, num_edges).transpose(2, 0, 1)
    return message_, alpha


# speedup vs baseline: 1.3229x; 1.0374x over previous
import functools

import jax
import jax.numpy as jnp
from jax import lax
from jax.experimental import pallas as pl
from jax.experimental.pallas import tpu as pltpu
from jax.experimental.pallas import tpu_sc as plsc

NUM_HEADS = 8
HEAD_DIM = 16
DIM = NUM_HEADS * HEAD_DIM

NC = 2
NS = 16
NW = NC * NS

CHUNK = 80


def _mm_body(x_ref, w_ref, o_ref):
    m = lax.dot_general(x_ref[...], w_ref[...], (((2,), (0,)), ((), ())),
                        preferred_element_type=jnp.float32)
    o_ref[...] = m.reshape(o_ref.shape)


def _score_body(msg_ref, w_ref, g_ref, v_ref, mt_ref):
    msg = msg_ref[...]
    m = lax.dot_general(msg, w_ref[...], (((2,), (0,)), ((), ())),
                        preferred_element_type=jnp.float32)
    x = m.reshape(g_ref.shape) + g_ref[...]
    x = jnp.maximum(x, 0.01 * x)
    v_ref[...] = jnp.exp(x)
    msg2 = msg.reshape(msg.shape[0] * HEAD_DIM, DIM)
    mt_ref[...] = msg2.T


def _add_body(p_ref, o_ref):
    o_ref[...] = p_ref[0] + p_ref[1]


def _div_body(v_ref, d_ref, o_ref):
    a = v_ref[...] / (d_ref[...] + 1e-16)
    rb = a.shape[0]
    a4 = a.reshape(rb // 8, 8, HEAD_DIM, NUM_HEADS)
    o_ref[...] = a4.transpose(3, 0, 1, 2).reshape(NUM_HEADS, rb * HEAD_DIM)


def kernel(edge_index, message, x_e, weight):
    num_edges = message.shape[0]
    num_nodes = x_e.shape[0]
    target = edge_index[1]

    w_m = weight[:, :HEAD_DIM]
    w_x = weight[:, HEAD_DIM:]
    eye = jnp.eye(NUM_HEADS, dtype=jnp.float32)
    W1 = (w_m[:, :, None] * eye[:, None, :]).reshape(DIM, NUM_HEADS)
    W2 = (w_x[:, :, None] * eye[:, None, :]).reshape(DIM, NUM_HEADS)

    n_chunks = num_edges // CHUNK
    rpw = n_chunks // NW
    nodes_pad = ((num_nodes + 16 * NS - 1) // (16 * NS)) * (16 * NS)
    rows_per_tile = nodes_pad // NS
    lanes = 128
    eflat_rows = num_edges * NUM_HEADS // lanes
    nflat_rows = num_nodes * NUM_HEADS // lanes

    t3d = target.reshape(NW, rpw, CHUNK)

    s_flat = pl.pallas_call(
        _mm_body,
        in_specs=[pl.BlockSpec((nflat_rows, HEAD_DIM, DIM), lambda: (0, 0, 0)),
                  pl.BlockSpec((DIM, NUM_HEADS), lambda: (0, 0))],
        out_specs=pl.BlockSpec((nflat_rows, lanes), lambda: (0, 0)),
        out_shape=jax.ShapeDtypeStruct((nflat_rows, lanes), jnp.float32),
    )(x_e.reshape(nflat_rows, HEAD_DIM, DIM), W2)
    s = s_flat.reshape(num_nodes, NUM_HEADS)

    mesh = plsc.VectorSubcoreMesh(core_axis_name="c", subcore_axis_name="s")

    @functools.partial(
        pl.kernel, mesh=mesh,
        compiler_params=pltpu.CompilerParams(use_tc_tiling_on_sc=False),
        out_type=jax.ShapeDtypeStruct((n_chunks, CHUNK, NUM_HEADS),
                                      jnp.float32),
        scratch_types=[
            pltpu.VMEM((rpw, CHUNK), jnp.int32),
            pltpu.VMEM((rpw, CHUNK, NUM_HEADS), jnp.float32),
            pltpu.SemaphoreType.DMA,
        ],
    )
    def gather_k(s_hbm, t_hbm, g_hbm, idx_v, big_v, sem):
        wid = lax.axis_index("c") * NS + lax.axis_index("s")
        row0 = wid * rpw
        pltpu.sync_copy(t_hbm.at[wid], idx_v)

        def body(j, carry):
            pltpu.async_copy(s_hbm.at[idx_v.at[j]], big_v.at[j], sem)
            return carry

        lax.fori_loop(0, rpw, body, 0)
        pltpu.make_async_copy(g_hbm.at[pl.ds(row0, rpw)], big_v, sem).wait()
        pltpu.sync_copy(big_v, g_hbm.at[pl.ds(row0, rpw)])

    g_flat = gather_k(s, t3d).reshape(eflat_rows, lanes)

    eb = 12800
    fb = eb * NUM_HEADS // lanes
    v_flat, msg_t = pl.pallas_call(
        _score_body,
        grid=(num_edges // eb,),
        in_specs=[pl.BlockSpec((fb, HEAD_DIM, DIM), lambda i: (i, 0, 0)),
                  pl.BlockSpec((DIM, NUM_HEADS), lambda i: (0, 0)),
                  pl.BlockSpec((fb, lanes), lambda i: (i, 0))],
        out_specs=[pl.BlockSpec((fb, lanes), lambda i: (i, 0)),
                   pl.BlockSpec((DIM, eb), lambda i: (0, i))],
        out_shape=[jax.ShapeDtypeStruct((eflat_rows, lanes), jnp.float32),
                   jax.ShapeDtypeStruct((DIM, num_edges), jnp.float32)],
    )(message.reshape(eflat_rows, HEAD_DIM, DIM), W1, g_flat)

    v3d = v_flat.reshape(n_chunks, CHUNK, NUM_HEADS)

    zeros_tab = jnp.zeros((nodes_pad, NUM_HEADS), jnp.float32)

    @functools.partial(
        pl.kernel, mesh=mesh,
        compiler_params=pltpu.CompilerParams(use_tc_tiling_on_sc=False),
        out_type=jax.ShapeDtypeStruct((NC, nodes_pad, NUM_HEADS), jnp.float32),
        scratch_types=[
            pltpu.VMEM((rpw, CHUNK), jnp.int32),
            pltpu.VMEM((rpw, CHUNK, NUM_HEADS), jnp.float32),
            pltpu.VMEM_SHARED((nodes_pad, NUM_HEADS), jnp.float32),
            pltpu.SemaphoreType.DMA,
            pltpu.SemaphoreType.DMA,
        ],
    )
    def segsum_k(v_hbm, t_hbm, z_hbm, part_hbm, idx_v, big_v, seg_sh,
                 sem, sem2):
        c = lax.axis_index("c")
        sid = lax.axis_index("s")
        wid = c * NS + sid
        row0 = wid * rpw

        pltpu.sync_copy(z_hbm.at[pl.ds(sid * rows_per_tile, rows_per_tile)],
                        seg_sh.at[pl.ds(sid * rows_per_tile, rows_per_tile)])
        pltpu.sync_copy(t_hbm.at[wid], idx_v)
        pltpu.sync_copy(v_hbm.at[pl.ds(row0, rpw)], big_v)
        plsc.subcore_barrier()

        def body(j, carry):
            pltpu.async_copy(big_v.at[j], seg_sh.at[idx_v.at[j]], sem2,
                             add=True)
            return carry

        lax.fori_loop(0, rpw, body, 0)
        pltpu.make_async_copy(v_hbm.at[pl.ds(row0, rpw)], big_v, sem2).wait()
        plsc.subcore_barrier()

        pltpu.sync_copy(seg_sh.at[pl.ds(sid * rows_per_tile, rows_per_tile)],
                        part_hbm.at[c, pl.ds(sid * rows_per_tile,
                                             rows_per_tile)])

    partial_tabs = segsum_k(v3d, t3d, zeros_tab)

    tab_rows = nodes_pad * NUM_HEADS // lanes
    seg_flat = pl.pallas_call(
        _add_body,
        in_specs=[pl.BlockSpec((NC, tab_rows, lanes), lambda: (0, 0, 0))],
        out_specs=pl.BlockSpec((tab_rows, lanes), lambda: (0, 0)),
        out_shape=jax.ShapeDtypeStruct((tab_rows, lanes), jnp.float32),
    )(partial_tabs.reshape(NC, tab_rows, lanes))
    seg_tab = seg_flat.reshape(nodes_pad, NUM_HEADS)

    @functools.partial(
        pl.kernel, mesh=mesh,
        compiler_params=pltpu.CompilerParams(use_tc_tiling_on_sc=False),
        out_type=jax.ShapeDtypeStruct((n_chunks, CHUNK, NUM_HEADS),
                                      jnp.float32),
        scratch_types=[
            pltpu.VMEM((rpw, CHUNK), jnp.int32),
            pltpu.VMEM((rpw, CHUNK, NUM_HEADS), jnp.float32),
            pltpu.SemaphoreType.DMA,
        ],
    )
    def denom_k(p_hbm, t_hbm, d_hbm, idx_v, big_v, sem):
        wid = lax.axis_index("c") * NS + lax.axis_index("s")
        row0 = wid * rpw
        pltpu.sync_copy(t_hbm.at[wid], idx_v)

        def body(j, carry):
            pltpu.async_copy(p_hbm.at[idx_v.at[j]], big_v.at[j], sem)
            return carry

        lax.fori_loop(0, rpw, body, 0)
        pltpu.make_async_copy(d_hbm.at[pl.ds(row0, rpw)], big_v, sem).wait()
        pltpu.sync_copy(big_v, d_hbm.at[pl.ds(row0, rpw)])

    d_flat = denom_k(seg_tab, t3d).reshape(eflat_rows, lanes)

    db = 4000
    alpha_t = pl.pallas_call(
        _div_body,
        grid=(eflat_rows // db,),
        in_specs=[pl.BlockSpec((db, lanes), lambda i: (i, 0))] * 2,
        out_specs=pl.BlockSpec((NUM_HEADS, db * lanes // NUM_HEADS),
                               lambda i: (0, i)),
        out_shape=jax.ShapeDtypeStruct((NUM_HEADS, num_edges), jnp.float32),
    )(v_flat, d_flat)

    alpha = alpha_t.T
    message_ = msg_t.reshape(NUM_HEADS, HEAD_DIM, num_edges).transpose(2, 0, 1)
    return message_, alpha


# speedup vs baseline: 1.3620x; 1.0296x over previous
import functools

import jax
import jax.numpy as jnp
from jax import lax
from jax.experimental import pallas as pl
from jax.experimental.pallas import tpu as pltpu
from jax.experimental.pallas import tpu_sc as plsc

NUM_HEADS = 8
HEAD_DIM = 16
DIM = NUM_HEADS * HEAD_DIM

NC = 2
NS = 16
NW = NC * NS

CHUNK = 80


def _mm_body(x_ref, w_ref, ei_ref, o_ref, t_ref):
    m = lax.dot_general(x_ref[...], w_ref[...], (((2,), (0,)), ((), ())),
                        preferred_element_type=jnp.float32)
    o_ref[...] = m.reshape(o_ref.shape)
    t_ref[...] = ei_ref[1]


def _score_body(msg_ref, w_ref, g_ref, v_ref, mt_ref):
    msg = msg_ref[...]
    m = lax.dot_general(msg, w_ref[...], (((2,), (0,)), ((), ())),
                        preferred_element_type=jnp.float32)
    x = m.reshape(g_ref.shape) + g_ref[...]
    x = jnp.maximum(x, 0.01 * x)
    v_ref[...] = jnp.exp(x)
    msg2 = msg.reshape(msg.shape[0] * HEAD_DIM, DIM)
    mt_ref[...] = msg2.T


def _add_body(p_ref, o_ref):
    o_ref[...] = p_ref[0] + p_ref[1]


def _div_body(v_ref, d_ref, o_ref):
    a = v_ref[...] / (d_ref[...] + 1e-16)
    rb = a.shape[0]
    a4 = a.reshape(rb // 8, 8, HEAD_DIM, NUM_HEADS)
    o_ref[...] = a4.transpose(3, 0, 1, 2).reshape(NUM_HEADS, rb * HEAD_DIM)


def kernel(edge_index, message, x_e, weight):
    num_edges = message.shape[0]
    num_nodes = x_e.shape[0]

    w_m = weight[:, :HEAD_DIM]
    w_x = weight[:, HEAD_DIM:]
    eye = jnp.eye(NUM_HEADS, dtype=jnp.float32)
    W1 = (w_m[:, :, None] * eye[:, None, :]).reshape(DIM, NUM_HEADS)
    W2 = (w_x[:, :, None] * eye[:, None, :]).reshape(DIM, NUM_HEADS)

    n_chunks = num_edges // CHUNK
    rpw = n_chunks // NW
    nodes_pad = ((num_nodes + 16 * NS - 1) // (16 * NS)) * (16 * NS)
    rows_per_tile = nodes_pad // NS
    lanes = 128
    eflat_rows = num_edges * NUM_HEADS // lanes
    nflat_rows = num_nodes * NUM_HEADS // lanes

    ei_rows = num_edges // lanes

    s_flat, t_flat = pl.pallas_call(
        _mm_body,
        in_specs=[pl.BlockSpec((nflat_rows, HEAD_DIM, DIM), lambda: (0, 0, 0)),
                  pl.BlockSpec((DIM, NUM_HEADS), lambda: (0, 0)),
                  pl.BlockSpec((2, ei_rows, lanes), lambda: (0, 0, 0))],
        out_specs=[pl.BlockSpec((nflat_rows, lanes), lambda: (0, 0)),
                   pl.BlockSpec((ei_rows, lanes), lambda: (0, 0))],
        out_shape=[jax.ShapeDtypeStruct((nflat_rows, lanes), jnp.float32),
                   jax.ShapeDtypeStruct((ei_rows, lanes), jnp.int32)],
    )(x_e.reshape(nflat_rows, HEAD_DIM, DIM), W2,
      edge_index.reshape(2, ei_rows, lanes))
    s = s_flat.reshape(num_nodes, NUM_HEADS)
    t3d = t_flat.reshape(NW, rpw, CHUNK)

    mesh = plsc.VectorSubcoreMesh(core_axis_name="c", subcore_axis_name="s")

    @functools.partial(
        pl.kernel, mesh=mesh,
        compiler_params=pltpu.CompilerParams(use_tc_tiling_on_sc=False),
        out_type=jax.ShapeDtypeStruct((n_chunks, CHUNK, NUM_HEADS),
                                      jnp.float32),
        scratch_types=[
            pltpu.VMEM((rpw, CHUNK), jnp.int32),
            pltpu.VMEM((rpw, CHUNK, NUM_HEADS), jnp.float32),
            pltpu.SemaphoreType.DMA,
        ],
    )
    def gather_k(s_hbm, t_hbm, g_hbm, idx_v, big_v, sem):
        wid = lax.axis_index("c") * NS + lax.axis_index("s")
        row0 = wid * rpw
        pltpu.sync_copy(t_hbm.at[wid], idx_v)

        def body(j, carry):
            pltpu.async_copy(s_hbm.at[idx_v.at[j]], big_v.at[j], sem)
            return carry

        lax.fori_loop(0, rpw, body, 0)
        pltpu.make_async_copy(g_hbm.at[pl.ds(row0, rpw)], big_v, sem).wait()
        pltpu.sync_copy(big_v, g_hbm.at[pl.ds(row0, rpw)])

    g_flat = gather_k(s, t3d).reshape(eflat_rows, lanes)

    eb = 12800
    fb = eb * NUM_HEADS // lanes
    v_flat, msg_t = pl.pallas_call(
        _score_body,
        grid=(num_edges // eb,),
        in_specs=[pl.BlockSpec((fb, HEAD_DIM, DIM), lambda i: (i, 0, 0)),
                  pl.BlockSpec((DIM, NUM_HEADS), lambda i: (0, 0)),
                  pl.BlockSpec((fb, lanes), lambda i: (i, 0))],
        out_specs=[pl.BlockSpec((fb, lanes), lambda i: (i, 0)),
                   pl.BlockSpec((DIM, eb), lambda i: (0, i))],
        out_shape=[jax.ShapeDtypeStruct((eflat_rows, lanes), jnp.float32),
                   jax.ShapeDtypeStruct((DIM, num_edges), jnp.float32)],
    )(message.reshape(eflat_rows, HEAD_DIM, DIM), W1, g_flat)

    v3d = v_flat.reshape(n_chunks, CHUNK, NUM_HEADS)

    zeros_tab = jnp.zeros((nodes_pad, NUM_HEADS), jnp.float32)

    @functools.partial(
        pl.kernel, mesh=mesh,
        compiler_params=pltpu.CompilerParams(use_tc_tiling_on_sc=False),
        out_type=jax.ShapeDtypeStruct((NC, nodes_pad, NUM_HEADS), jnp.float32),
        scratch_types=[
            pltpu.VMEM((rpw, CHUNK), jnp.int32),
            pltpu.VMEM((rpw, CHUNK, NUM_HEADS), jnp.float32),
            pltpu.VMEM_SHARED((nodes_pad, NUM_HEADS), jnp.float32),
            pltpu.SemaphoreType.DMA,
            pltpu.SemaphoreType.DMA,
        ],
    )
    def segsum_k(v_hbm, t_hbm, z_hbm, part_hbm, idx_v, big_v, seg_sh,
                 sem, sem2):
        c = lax.axis_index("c")
        sid = lax.axis_index("s")
        wid = c * NS + sid
        row0 = wid * rpw

        pltpu.sync_copy(z_hbm.at[pl.ds(sid * rows_per_tile, rows_per_tile)],
                        seg_sh.at[pl.ds(sid * rows_per_tile, rows_per_tile)])
        pltpu.sync_copy(t_hbm.at[wid], idx_v)
        pltpu.sync_copy(v_hbm.at[pl.ds(row0, rpw)], big_v)
        plsc.subcore_barrier()

        def body(j, carry):
            pltpu.async_copy(big_v.at[j], seg_sh.at[idx_v.at[j]], sem2,
                             add=True)
            return carry

        lax.fori_loop(0, rpw, body, 0)
        pltpu.make_async_copy(v_hbm.at[pl.ds(row0, rpw)], big_v, sem2).wait()
        plsc.subcore_barrier()

        pltpu.sync_copy(seg_sh.at[pl.ds(sid * rows_per_tile, rows_per_tile)],
                        part_hbm.at[c, pl.ds(sid * rows_per_tile,
                                             rows_per_tile)])

    partial_tabs = segsum_k(v3d, t3d, zeros_tab)

    tab_rows = nodes_pad * NUM_HEADS // lanes
    seg_flat = pl.pallas_call(
        _add_body,
        in_specs=[pl.BlockSpec((NC, tab_rows, lanes), lambda: (0, 0, 0))],
        out_specs=pl.BlockSpec((tab_rows, lanes), lambda: (0, 0)),
        out_shape=jax.ShapeDtypeStruct((tab_rows, lanes), jnp.float32),
    )(partial_tabs.reshape(NC, tab_rows, lanes))
    seg_tab = seg_flat.reshape(nodes_pad, NUM_HEADS)

    @functools.partial(
        pl.kernel, mesh=mesh,
        compiler_params=pltpu.CompilerParams(use_tc_tiling_on_sc=False),
        out_type=jax.ShapeDtypeStruct((n_chunks, CHUNK, NUM_HEADS),
                                      jnp.float32),
        scratch_types=[
            pltpu.VMEM((rpw, CHUNK), jnp.int32),
            pltpu.VMEM((rpw, CHUNK, NUM_HEADS), jnp.float32),
            pltpu.SemaphoreType.DMA,
        ],
    )
    def denom_k(p_hbm, t_hbm, d_hbm, idx_v, big_v, sem):
        wid = lax.axis_index("c") * NS + lax.axis_index("s")
        row0 = wid * rpw
        pltpu.sync_copy(t_hbm.at[wid], idx_v)

        def body(j, carry):
            pltpu.async_copy(p_hbm.at[idx_v.at[j]], big_v.at[j], sem)
            return carry

        lax.fori_loop(0, rpw, body, 0)
        pltpu.make_async_copy(d_hbm.at[pl.ds(row0, rpw)], big_v, sem).wait()
        pltpu.sync_copy(big_v, d_hbm.at[pl.ds(row0, rpw)])

    d_flat = denom_k(seg_tab, t3d).reshape(eflat_rows, lanes)

    db = 4000
    alpha_t = pl.pallas_call(
        _div_body,
        grid=(eflat_rows // db,),
        in_specs=[pl.BlockSpec((db, lanes), lambda i: (i, 0))] * 2,
        out_specs=pl.BlockSpec((NUM_HEADS, db * lanes // NUM_HEADS),
                               lambda i: (0, i)),
        out_shape=jax.ShapeDtypeStruct((NUM_HEADS, num_edges), jnp.float32),
    )(v_flat, d_flat)

    alpha = alpha_t.T
    message_ = msg_t.reshape(NUM_HEADS, HEAD_DIM, num_edges).transpose(2, 0, 1)
    return message_, alpha
